# baseline (speedup 1.0000x reference)
import functools

import jax
import jax.numpy as jnp
from jax.experimental import pallas as pl
from jax.experimental.pallas import tpu as pltpu

_EMA = 0.99
_THRESHOLD = 0.7


def _dot(a, b, dims):
    return jax.lax.dot_general(a, b, (dims, ((), ())),
                               precision=jax.lax.Precision.DEFAULT,
                               preferred_element_type=jnp.float32)


def _gelu(x):
    return 0.5 * x * (1.0 + jax.lax.erf(x * (2.0 ** -0.5)))


def _head_body(x_ref, W1_ref, W2_ref, b1_ref, b2_ref, rmsw_ref,
               We1_ref, be1_ref, We2_ref, be2_ref, WgT_ref, bg_ref,
               cent_ref, prec_ref,
               scale_ref, ood_ref, mah_ref, en_ref, su_ref,
               acc_ref, u1_ref, v1_ref, u2_ref, v2_ref, *, nsteps, L,
               n_iter=8):
    i = pl.program_id(0)

    @pl.when(i == 0)
    def _init():
        acc_ref[...] = jnp.zeros_like(acc_ref)
        u1_ref[...] = jnp.full_like(u1_ref, 1.0 / (u1_ref.shape[1] ** 0.5))
        u2_ref[...] = jnp.full_like(u2_ref, 1.0 / (u2_ref.shape[1] ** 0.5))

    iters_per_step = -(-n_iter // nsteps)
    @pl.when(i * iters_per_step < n_iter)
    def _power_step():
        for _ in range(iters_per_step):
            for W_ref, u_ref, v_ref in ((W1_ref, u1_ref, v1_ref),
                                        (W2_ref, u2_ref, v2_ref)):
                W = W_ref[...]
                v = _dot(u_ref[...], W, ((1,), (0,)))
                v = v / (jnp.sqrt(jnp.sum(v * v)) + 1e-12)
                u = _dot(v, W, ((1,), (1,)))
                u = u / (jnp.sqrt(jnp.sum(u * u)) + 1e-12)
                u_ref[...] = u
                v_ref[...] = v

    acc_ref[...] += jnp.sum(x_ref[...], axis=1)

    @pl.when(i == nsteps - 1)
    def _head():
        B = acc_ref.shape[0]
        K = cent_ref.shape[0]
        pooled = acc_ref[...] * (1.0 / L)

        s1 = jnp.sum(u1_ref[...] * _dot(v1_ref[...], W1_ref[...],
                                        ((1,), (1,))))
        s2 = jnp.sum(u2_ref[...] * _dot(v2_ref[...], W2_ref[...],
                                        ((1,), (1,))))
        W1n = W1_ref[...] / s1
        W2n = W2_ref[...] / s2
        h1 = _gelu(_dot(pooled, W1n, ((1,), (1,))) + b1_ref[...])
        f_pre = _dot(h1, W2n, ((1,), (1,))) + b2_ref[...]
        rms = jax.lax.rsqrt(jnp.mean(f_pre * f_pre, axis=-1, keepdims=True)
                            + 1e-6)
        feat = f_pre * rms * rmsw_ref[...]

        cent = cent_ref[...]
        cols = []
        for b in range(B):
            diff = cent - feat[b:b + 1, :]
            cols.append(jnp.sum(diff * diff, axis=1, keepdims=True))
        d2T = jnp.concatenate(cols, axis=1)
        dmin = jnp.min(d2T, axis=0, keepdims=True)
        iotaK = jax.lax.broadcasted_iota(jnp.int32, (K, B), 0)
        cand = jnp.where(d2T == dmin, iotaK, K)
        nearestT = jnp.min(cand, axis=0, keepdims=True)
        onehotT = (iotaK == nearestT).astype(jnp.float32)
        countsK = jnp.sum(onehotT, axis=1, keepdims=True)
        sums = _dot(onehotT, feat, ((1,), (0,)))
        bmean = sums / jnp.maximum(countsK, 1.0)
        cent_new = jnp.where(countsK > 0.0,
                             _EMA * cent + (1.0 - _EMA) * bmean, cent)

        prec = prec_ref[...]
        be2s = jnp.sum(be2_ref[...])
        g1 = _gelu(_dot(feat, We1_ref[...], ((1,), (1,))) + be1_ref[...])
        g1b = g1.astype(jnp.bfloat16).astype(jnp.float32)
        We2 = We2_ref[...].astype(jnp.bfloat16).astype(jnp.float32)

        mah_s, en_s, nrm_s = [], [], []
        for b in range(B):
            diff = cent_new - feat[b:b + 1, :]
            m = jnp.sum(diff * diff * prec, axis=1, keepdims=True)
            mah_s.append(jnp.sqrt(jnp.min(m)))
            en_s.append(jax.nn.sigmoid(jnp.sum(g1b[b:b + 1, :] * We2) + be2s))
            nrm_s.append(jnp.sqrt(jnp.sum(feat[b:b + 1, :] ** 2)))

        mah_max = functools.reduce(jnp.maximum, mah_s)
        nrm_max = functools.reduce(jnp.maximum, nrm_s)
        WgT = WgT_ref[...]
        bg = bg_ref[...]
        for b in range(B):
            su_b = 1.0 - nrm_s[b] / (nrm_max + 1e-6)
            ood_b = (mah_s[b] / (mah_max + 1e-6) + en_s[b] + su_b) / 3.0
            gate = jax.nn.sigmoid(ood_b * WgT + bg)
            scale_ref[b:b + 1, :] = 0.7 + 0.3 * gate
            ood_ref[b:b + 1, :] = jnp.full((1, 1), ood_b, jnp.float32)
            mah_ref[b:b + 1, :] = jnp.full((1, 1), mah_s[b], jnp.float32)
            en_ref[b:b + 1, :] = jnp.full((1, 1), en_s[b], jnp.float32)
            su_ref[b:b + 1, :] = jnp.full((1, 1), su_b, jnp.float32)


def _scale_body(x_ref, scale_ref, out_ref):
    out_ref[...] = x_ref[...] * scale_ref[...][:, None, :]


@jax.jit
def kernel(x, W1, b1, W2, b2, rms_w, We1, be1, We2, be2, Wg, bg,
           centroids, precision_diag):
    B, L, D = x.shape
    H = W1.shape[0]
    Hh = We1.shape[0]
    K = centroids.shape[0]
    LC = 256
    LCP = 512
    nsteps = L // LC
    npool = L // LCP

    full = lambda shape: pl.BlockSpec(shape, lambda i: (0,) * len(shape))

    head = pl.pallas_call(
        functools.partial(_head_body, nsteps=npool, L=L),
        grid=(npool,),
        in_specs=[
            pl.BlockSpec((B, LCP, D), lambda i: (0, i, 0)),
            full((H, D)), full((H, H)),
            full((1, H)), full((1, H)), full((1, H)),
            full((Hh, H)), full((1, Hh)), full((1, Hh)), full((1, 1)),
            full((1, D)), full((1, D)),
            full((K, H)), full((1, H)),
        ],
        out_specs=[
            full((B, D)),
            full((B, 1)), full((B, 1)), full((B, 1)), full((B, 1)),
        ],
        out_shape=[
            jax.ShapeDtypeStruct((B, D), jnp.float32),
            jax.ShapeDtypeStruct((B, 1), jnp.float32),
            jax.ShapeDtypeStruct((B, 1), jnp.float32),
            jax.ShapeDtypeStruct((B, 1), jnp.float32),
            jax.ShapeDtypeStruct((B, 1), jnp.float32),
        ],
        scratch_shapes=[pltpu.VMEM((B, D), jnp.float32),
                        pltpu.VMEM((1, H), jnp.float32),
                        pltpu.VMEM((1, D), jnp.float32),
                        pltpu.VMEM((1, H), jnp.float32),
                        pltpu.VMEM((1, H), jnp.float32)],
    )

    scale, ood, mah, en, su = head(
        x, W1, W2,
        b1.reshape(1, H), b2.reshape(1, H), rms_w.reshape(1, H),
        We1, be1.reshape(1, Hh), We2, be2.reshape(1, 1),
        Wg.reshape(1, D), bg.reshape(1, D),
        centroids, precision_diag.reshape(1, H),
    )

    x_ood = pl.pallas_call(
        _scale_body,
        grid=(nsteps,),
        in_specs=[
            pl.BlockSpec((B, LC, D), lambda i: (0, i, 0)),
            pl.BlockSpec((B, D), lambda i: (0, 0)),
        ],
        out_specs=pl.BlockSpec((B, LC, D), lambda i: (0, i, 0)),
        out_shape=jax.ShapeDtypeStruct((B, L, D), jnp.float32),
        compiler_params=pltpu.CompilerParams(
            dimension_semantics=("arbitrary",)),
    )(x, scale)

    ood_score = ood.reshape(B)
    return (x_ood, ood_score, ood_score > _THRESHOLD, mah.reshape(B),
            en.reshape(B), su.reshape(B))


# speedup vs baseline: 1.0999x; 1.0999x over previous
import functools

import jax
import jax.numpy as jnp
from jax.experimental import pallas as pl
from jax.experimental.pallas import tpu as pltpu

_EMA = 0.99
_THRESHOLD = 0.7


def _dot(a, b, dims):
    return jax.lax.dot_general(a, b, (dims, ((), ())),
                               precision=jax.lax.Precision.DEFAULT,
                               preferred_element_type=jnp.float32)


def _gelu(x):
    return 0.5 * x * (1.0 + jax.lax.erf(x * (2.0 ** -0.5)))


def _fused_body(x_ref, xt_ref, W1_ref, W2_ref, b1_ref, b2_ref, rmsw_ref,
                We1_ref, be1_ref, We2_ref, be2_ref, WgT_ref, bg_ref,
                cent_ref, prec_ref,
                out_ref, ood_ref, mah_ref, en_ref, su_ref,
                acc_ref, scale_ref, u1_ref, v1_ref, u2_ref, v2_ref,
                *, nsteps, L, LC, n_iter=8):
    p = pl.program_id(0)
    i = pl.program_id(1)
    half = nsteps // 2

    @pl.when((p == 0) & (i == 0))
    def _init():
        acc_ref[...] = jnp.zeros_like(acc_ref)
        u1_ref[...] = jnp.full_like(u1_ref, 1.0 / (u1_ref.shape[1] ** 0.5))
        u2_ref[...] = jnp.full_like(u2_ref, 1.0 / (u2_ref.shape[1] ** 0.5))

    @pl.when((p == 0) & (i < n_iter))
    def _power_step():
        for W_ref, u_ref, v_ref in ((W1_ref, u1_ref, v1_ref),
                                    (W2_ref, u2_ref, v2_ref)):
            W = W_ref[...]
            v = _dot(u_ref[...], W, ((1,), (0,)))
            v = v / (jnp.sqrt(jnp.sum(v * v)) + 1e-12)
            u = _dot(v, W, ((1,), (1,)))
            u = u / (jnp.sqrt(jnp.sum(u * u)) + 1e-12)
            u_ref[...] = u
            v_ref[...] = v

    @pl.when((p == 0) & (i < half))
    def _pool_stream():
        acc_ref[...] += jnp.sum(x_ref[...], axis=1)

    @pl.when((p == 0) & (i >= half))
    def _pool_retained():
        acc_ref[...] += jnp.sum(
            xt_ref[:, pl.ds((i - half) * LC, LC), :], axis=1)

    @pl.when((p == 0) & (i == nsteps - 1))
    def _head():
        B = acc_ref.shape[0]
        K = cent_ref.shape[0]
        pooled = acc_ref[...] * (1.0 / L)

        s1 = jnp.sum(u1_ref[...] * _dot(v1_ref[...], W1_ref[...],
                                        ((1,), (1,))))
        s2 = jnp.sum(u2_ref[...] * _dot(v2_ref[...], W2_ref[...],
                                        ((1,), (1,))))
        W1n = W1_ref[...] / s1
        W2n = W2_ref[...] / s2
        h1 = _gelu(_dot(pooled, W1n, ((1,), (1,))) + b1_ref[...])
        f_pre = _dot(h1, W2n, ((1,), (1,))) + b2_ref[...]
        rms = jax.lax.rsqrt(jnp.mean(f_pre * f_pre, axis=-1, keepdims=True)
                            + 1e-6)
        feat = f_pre * rms * rmsw_ref[...]

        cent = cent_ref[...]
        cols = []
        for b in range(B):
            diff = cent - feat[b:b + 1, :]
            cols.append(jnp.sum(diff * diff, axis=1, keepdims=True))
        d2T = jnp.concatenate(cols, axis=1)
        dmin = jnp.min(d2T, axis=0, keepdims=True)
        iotaK = jax.lax.broadcasted_iota(jnp.int32, (K, B), 0)
        cand = jnp.where(d2T == dmin, iotaK, K)
        nearestT = jnp.min(cand, axis=0, keepdims=True)
        onehotT = (iotaK == nearestT).astype(jnp.float32)
        countsK = jnp.sum(onehotT, axis=1, keepdims=True)
        sums = _dot(onehotT, feat, ((1,), (0,)))
        bmean = sums / jnp.maximum(countsK, 1.0)
        cent_new = jnp.where(countsK > 0.0,
                             _EMA * cent + (1.0 - _EMA) * bmean, cent)

        prec = prec_ref[...]
        be2s = jnp.sum(be2_ref[...])
        g1 = _gelu(_dot(feat, We1_ref[...], ((1,), (1,))) + be1_ref[...])
        g1b = g1.astype(jnp.bfloat16).astype(jnp.float32)
        We2 = We2_ref[...].astype(jnp.bfloat16).astype(jnp.float32)

        mah_s, en_s, nrm_s = [], [], []
        for b in range(B):
            diff = cent_new - feat[b:b + 1, :]
            m = jnp.sum(diff * diff * prec, axis=1, keepdims=True)
            mah_s.append(jnp.sqrt(jnp.min(m)))
            en_s.append(jax.nn.sigmoid(jnp.sum(g1b[b:b + 1, :] * We2) + be2s))
            nrm_s.append(jnp.sqrt(jnp.sum(feat[b:b + 1, :] ** 2)))

        mah_max = functools.reduce(jnp.maximum, mah_s)
        nrm_max = functools.reduce(jnp.maximum, nrm_s)
        WgT = WgT_ref[...]
        bg = bg_ref[...]
        for b in range(B):
            su_b = 1.0 - nrm_s[b] / (nrm_max + 1e-6)
            ood_b = (mah_s[b] / (mah_max + 1e-6) + en_s[b] + su_b) / 3.0
            gate = jax.nn.sigmoid(ood_b * WgT + bg)
            scale_ref[b:b + 1, :] = 0.7 + 0.3 * gate
            ood_ref[b:b + 1, :] = jnp.full((1, 1), ood_b, jnp.float32)
            mah_ref[b:b + 1, :] = jnp.full((1, 1), mah_s[b], jnp.float32)
            en_ref[b:b + 1, :] = jnp.full((1, 1), en_s[b], jnp.float32)
            su_ref[b:b + 1, :] = jnp.full((1, 1), su_b, jnp.float32)

    @pl.when((p == 1) & (i < half))
    def _scale_stream():
        out_ref[...] = x_ref[...] * scale_ref[...][:, None, :]

    @pl.when((p == 1) & (i >= half))
    def _scale_retained():
        out_ref[...] = (xt_ref[:, pl.ds((i - half) * LC, LC), :]
                        * scale_ref[...][:, None, :])


@jax.jit
def kernel(x, W1, b1, W2, b2, rms_w, We1, be1, We2, be2, Wg, bg,
           centroids, precision_diag):
    B, L, D = x.shape
    H = W1.shape[0]
    Hh = We1.shape[0]
    K = centroids.shape[0]
    LC = 128
    nsteps = L // LC
    half = nsteps // 2
    Lt = L // 2

    full = lambda shape: pl.BlockSpec(shape, lambda p, i: (0,) * len(shape))

    out = pl.pallas_call(
        functools.partial(_fused_body, nsteps=nsteps, L=L, LC=LC),
        grid=(2, nsteps),
        in_specs=[
            pl.BlockSpec((B, LC, D),
                         lambda p, i: (0, jnp.minimum(i, half - 1), 0)),
            pl.BlockSpec((B, Lt, D), lambda p, i: (0, 1, 0)),
            full((H, D)), full((H, H)),
            full((1, H)), full((1, H)), full((1, H)),
            full((Hh, H)), full((1, Hh)), full((1, Hh)), full((1, 1)),
            full((1, D)), full((1, D)),
            full((K, H)), full((1, H)),
        ],
        out_specs=[
            pl.BlockSpec((B, LC, D), lambda p, i: (0, p * i, 0)),
            full((B, 1)), full((B, 1)), full((B, 1)), full((B, 1)),
        ],
        out_shape=[
            jax.ShapeDtypeStruct((B, L, D), jnp.float32),
            jax.ShapeDtypeStruct((B, 1), jnp.float32),
            jax.ShapeDtypeStruct((B, 1), jnp.float32),
            jax.ShapeDtypeStruct((B, 1), jnp.float32),
            jax.ShapeDtypeStruct((B, 1), jnp.float32),
        ],
        scratch_shapes=[pltpu.VMEM((B, D), jnp.float32),
                        pltpu.VMEM((B, D), jnp.float32),
                        pltpu.VMEM((1, H), jnp.float32),
                        pltpu.VMEM((1, D), jnp.float32),
                        pltpu.VMEM((1, H), jnp.float32),
                        pltpu.VMEM((1, H), jnp.float32)],
        compiler_params=pltpu.CompilerParams(
            dimension_semantics=("arbitrary", "arbitrary")),
    )(
        x, x, W1, W2,
        b1.reshape(1, H), b2.reshape(1, H), rms_w.reshape(1, H),
        We1, be1.reshape(1, Hh), We2, be2.reshape(1, 1),
        Wg.reshape(1, D), bg.reshape(1, D),
        centroids, precision_diag.reshape(1, H),
    )

    x_ood, ood, mah, en, su = out
    ood_score = ood.reshape(B)
    return (x_ood, ood_score, ood_score > _THRESHOLD, mah.reshape(B),
            en.reshape(B), su.reshape(B))


# speedup vs baseline: 1.1011x; 1.0011x over previous
import functools

import jax
import jax.numpy as jnp
from jax.experimental import pallas as pl
from jax.experimental.pallas import tpu as pltpu

_EMA = 0.99
_THRESHOLD = 0.7


def _dot(a, b, dims):
    return jax.lax.dot_general(a, b, (dims, ((), ())),
                               precision=jax.lax.Precision.DEFAULT,
                               preferred_element_type=jnp.float32)


def _gelu(x):
    return 0.5 * x * (1.0 + jax.lax.erf(x * (2.0 ** -0.5)))


def _fused_body(x_ref, xt_ref, W1_ref, W2_ref, b1_ref, b2_ref, rmsw_ref,
                We1_ref, be1_ref, We2_ref, be2_ref, WgT_ref, bg_ref,
                cent_ref, prec_ref,
                out_ref, ood_ref, mah_ref, en_ref, su_ref,
                acc_ref, scale_ref, u1_ref, v1_ref, u2_ref, v2_ref,
                *, nsteps, L, LC, n_iter=8):
    p = pl.program_id(0)
    i = pl.program_id(1)

    @pl.when((p == 0) & (i == 0))
    def _init():
        acc_ref[...] = jnp.zeros_like(acc_ref)
        u1_ref[...] = jnp.full_like(u1_ref, 1.0 / (u1_ref.shape[1] ** 0.5))
        u2_ref[...] = jnp.full_like(u2_ref, 1.0 / (u2_ref.shape[1] ** 0.5))

    @pl.when((p == 0) & (i < n_iter))
    def _power_step():
        for W_ref, u_ref, v_ref in ((W1_ref, u1_ref, v1_ref),
                                    (W2_ref, u2_ref, v2_ref)):
            W = W_ref[...]
            v = _dot(u_ref[...], W, ((1,), (0,)))
            v = v / (jnp.sqrt(jnp.sum(v * v)) + 1e-12)
            u = _dot(v, W, ((1,), (1,)))
            u = u / (jnp.sqrt(jnp.sum(u * u)) + 1e-12)
            u_ref[...] = u
            v_ref[...] = v

    @pl.when(p == 0)
    def _pool():
        acc_ref[...] += (jnp.sum(x_ref[...], axis=1)
                         + jnp.sum(xt_ref[:, pl.ds(i * LC, LC), :], axis=1))

    @pl.when((p == 0) & (i == nsteps - 1))
    def _head():
        B = acc_ref.shape[0]
        K = cent_ref.shape[0]
        pooled = acc_ref[...] * (1.0 / L)

        s1 = jnp.sum(u1_ref[...] * _dot(v1_ref[...], W1_ref[...],
                                        ((1,), (1,))))
        s2 = jnp.sum(u2_ref[...] * _dot(v2_ref[...], W2_ref[...],
                                        ((1,), (1,))))
        W1n = W1_ref[...] / s1
        W2n = W2_ref[...] / s2
        h1 = _gelu(_dot(pooled, W1n, ((1,), (1,))) + b1_ref[...])
        f_pre = _dot(h1, W2n, ((1,), (1,))) + b2_ref[...]
        rms = jax.lax.rsqrt(jnp.mean(f_pre * f_pre, axis=-1, keepdims=True)
                            + 1e-6)
        feat = f_pre * rms * rmsw_ref[...]

        cent = cent_ref[...]
        cols = []
        for b in range(B):
            diff = cent - feat[b:b + 1, :]
            cols.append(jnp.sum(diff * diff, axis=1, keepdims=True))
        d2T = jnp.concatenate(cols, axis=1)
        dmin = jnp.min(d2T, axis=0, keepdims=True)
        iotaK = jax.lax.broadcasted_iota(jnp.int32, (K, B), 0)
        cand = jnp.where(d2T == dmin, iotaK, K)
        nearestT = jnp.min(cand, axis=0, keepdims=True)
        onehotT = (iotaK == nearestT).astype(jnp.float32)
        countsK = jnp.sum(onehotT, axis=1, keepdims=True)
        sums = _dot(onehotT, feat, ((1,), (0,)))
        bmean = sums / jnp.maximum(countsK, 1.0)
        cent_new = jnp.where(countsK > 0.0,
                             _EMA * cent + (1.0 - _EMA) * bmean, cent)

        prec = prec_ref[...]
        be2s = jnp.sum(be2_ref[...])
        g1 = _gelu(_dot(feat, We1_ref[...], ((1,), (1,))) + be1_ref[...])
        g1b = g1.astype(jnp.bfloat16).astype(jnp.float32)
        We2 = We2_ref[...].astype(jnp.bfloat16).astype(jnp.float32)

        mah_s, en_s, nrm_s = [], [], []
        for b in range(B):
            diff = cent_new - feat[b:b + 1, :]
            m = jnp.sum(diff * diff * prec, axis=1, keepdims=True)
            mah_s.append(jnp.sqrt(jnp.min(m)))
            en_s.append(jax.nn.sigmoid(jnp.sum(g1b[b:b + 1, :] * We2) + be2s))
            nrm_s.append(jnp.sqrt(jnp.sum(feat[b:b + 1, :] ** 2)))

        mah_max = functools.reduce(jnp.maximum, mah_s)
        nrm_max = functools.reduce(jnp.maximum, nrm_s)
        WgT = WgT_ref[...]
        bg = bg_ref[...]
        for b in range(B):
            su_b = 1.0 - nrm_s[b] / (nrm_max + 1e-6)
            ood_b = (mah_s[b] / (mah_max + 1e-6) + en_s[b] + su_b) / 3.0
            gate = jax.nn.sigmoid(ood_b * WgT + bg)
            scale_ref[b:b + 1, :] = 0.7 + 0.3 * gate
            ood_ref[b:b + 1, :] = jnp.full((1, 1), ood_b, jnp.float32)
            mah_ref[b:b + 1, :] = jnp.full((1, 1), mah_s[b], jnp.float32)
            en_ref[b:b + 1, :] = jnp.full((1, 1), en_s[b], jnp.float32)
            su_ref[b:b + 1, :] = jnp.full((1, 1), su_b, jnp.float32)

    @pl.when(p == 1)
    def _scale_stream():
        out_ref[...] = x_ref[...] * scale_ref[...][:, None, :]

    @pl.when(p == 2)
    def _scale_retained():
        out_ref[...] = (xt_ref[:, pl.ds(i * LC, LC), :]
                        * scale_ref[...][:, None, :])


@jax.jit
def kernel(x, W1, b1, W2, b2, rms_w, We1, be1, We2, be2, Wg, bg,
           centroids, precision_diag):
    B, L, D = x.shape
    H = W1.shape[0]
    Hh = We1.shape[0]
    K = centroids.shape[0]
    Lt = L // 2
    nsteps = 8
    LC = Lt // nsteps

    full = lambda shape: pl.BlockSpec(shape, lambda p, i: (0,) * len(shape))

    out = pl.pallas_call(
        functools.partial(_fused_body, nsteps=nsteps, L=L, LC=LC),
        grid=(3, nsteps),
        in_specs=[
            pl.BlockSpec((B, LC, D),
                         lambda p, i: (0, jnp.where(p == 2, nsteps - 1, i),
                                       0)),
            pl.BlockSpec((B, Lt, D), lambda p, i: (0, 1, 0)),
            full((H, D)), full((H, H)),
            full((1, H)), full((1, H)), full((1, H)),
            full((Hh, H)), full((1, Hh)), full((1, Hh)), full((1, 1)),
            full((1, D)), full((1, D)),
            full((K, H)), full((1, H)),
        ],
        out_specs=[
            pl.BlockSpec((B, LC, D),
                         lambda p, i: (0, jnp.where(p == 0, 0,
                                                    (p - 1) * nsteps + i),
                                       0)),
            full((B, 1)), full((B, 1)), full((B, 1)), full((B, 1)),
        ],
        out_shape=[
            jax.ShapeDtypeStruct((B, L, D), jnp.float32),
            jax.ShapeDtypeStruct((B, 1), jnp.float32),
            jax.ShapeDtypeStruct((B, 1), jnp.float32),
            jax.ShapeDtypeStruct((B, 1), jnp.float32),
            jax.ShapeDtypeStruct((B, 1), jnp.float32),
        ],
        scratch_shapes=[pltpu.VMEM((B, D), jnp.float32),
                        pltpu.VMEM((B, D), jnp.float32),
                        pltpu.VMEM((1, H), jnp.float32),
                        pltpu.VMEM((1, D), jnp.float32),
                        pltpu.VMEM((1, H), jnp.float32),
                        pltpu.VMEM((1, H), jnp.float32)],
        compiler_params=pltpu.CompilerParams(
            dimension_semantics=("arbitrary", "arbitrary")),
    )(
        x, x, W1, W2,
        b1.reshape(1, H), b2.reshape(1, H), rms_w.reshape(1, H),
        We1, be1.reshape(1, Hh), We2, be2.reshape(1, 1),
        Wg.reshape(1, D), bg.reshape(1, D),
        centroids, precision_diag.reshape(1, H),
    )

    x_ood, ood, mah, en, su = out
    ood_score = ood.reshape(B)
    return (x_ood, ood_score, ood_score > _THRESHOLD, mah.reshape(B),
            en.reshape(B), su.reshape(B))
